# baseline (speedup 1.0000x reference)
import functools

import jax
import jax.numpy as jnp
from jax import lax
from jax.experimental import pallas as pl
from jax.experimental.pallas import tpu as pltpu
from jax.experimental.pallas import tpu_sc as plsc

N_NODES = 10000
EMB = 256
HALF = 128
N_EDGES = 160000

N_SUB = 16
EDGES_PER_TILE = N_EDGES // N_SUB
CHUNK = 80
N_CHUNKS = EDGES_PER_TILE // CHUNK
STAGES = ((0, 32), (32, 32), (64, 32), (96, 29))
STAGE_MAX = 32
NBUF = 4
ROWS_MAIN = 624
ROW_TAIL = N_NODES - N_SUB * ROWS_MAIN
ZROWS = 80
Z_FULL = 7
Z_REM = ROWS_MAIN - Z_FULL * ZROWS

MLP_BLK = 10000


def _mlp_body(x_ref, w1_ref, b1_ref, w2_ref, b2_ref, h0_ref, h1_ref):
    x16 = x_ref[...].astype(jnp.bfloat16)
    h = jnp.dot(x16, w1_ref[...].astype(jnp.bfloat16),
                preferred_element_type=jnp.float32)
    h = jnp.maximum(h + b1_ref[...], 0.0).astype(jnp.bfloat16)
    h = jnp.dot(h, w2_ref[...].astype(jnp.bfloat16),
                preferred_element_type=jnp.float32)
    h = jnp.maximum(h + b2_ref[...], 0.0)
    h0_ref[...] = h[:, :HALF]
    h1_ref[...] = h[:, HALF:]


def _mlp(x, W1, b1, W2, b2):
    return pl.pallas_call(
        _mlp_body,
        grid=(N_NODES // MLP_BLK,),
        in_specs=[
            pl.BlockSpec((MLP_BLK, EMB), lambda i: (i, 0)),
            pl.BlockSpec((EMB, EMB), lambda i: (0, 0)),
            pl.BlockSpec((1, EMB), lambda i: (0, 0)),
            pl.BlockSpec((EMB, EMB), lambda i: (0, 0)),
            pl.BlockSpec((1, EMB), lambda i: (0, 0)),
        ],
        out_specs=[
            pl.BlockSpec((MLP_BLK, HALF), lambda i: (i, 0)),
            pl.BlockSpec((MLP_BLK, HALF), lambda i: (i, 0)),
        ],
        out_shape=[
            jax.ShapeDtypeStruct((N_NODES, HALF), jnp.float32),
            jax.ShapeDtypeStruct((N_NODES, HALF), jnp.float32),
        ],
    )(x, W1, b1, W2, b2)


def _sc_message_pass(h0, h1, ei4):
    mesh = plsc.VectorSubcoreMesh(core_axis_name="c", subcore_axis_name="s")

    @functools.partial(
        pl.kernel,
        mesh=mesh,
        out_type=jax.ShapeDtypeStruct((N_NODES, EMB), jnp.float32),
        scratch_types=[
            pltpu.VMEM((STAGE_MAX, CHUNK), jnp.int32),
            pltpu.VMEM((STAGE_MAX, CHUNK), jnp.int32),
            pltpu.VMEM((CHUNK, HALF), jnp.float32),
            pltpu.VMEM((CHUNK, HALF), jnp.float32),
            pltpu.VMEM((CHUNK, HALF), jnp.float32),
            pltpu.VMEM((CHUNK, HALF), jnp.float32),
            pltpu.VMEM_SHARED((N_NODES, HALF), jnp.float32),
            pltpu.SemaphoreType.DMA,
            pltpu.SemaphoreType.DMA,
        ],
    )
    def k(h0_hbm, h1_hbm, ei_hbm, out_hbm, src_v, dst_v, b0, b1, b2, b3,
          acc, sem_g, sem_s):
        c = lax.axis_index("c")
        s = lax.axis_index("s")
        row0 = s * ROWS_MAIN
        bufs = (b0, b1, b2, b3)

        def run(h_ref):
            pltpu.sync_copy(ei_hbm.at[0, s, pl.ds(0, STAGES[0][1])],
                            src_v.at[pl.ds(0, STAGES[0][1])])
            pltpu.sync_copy(ei_hbm.at[1, s, pl.ds(0, STAGES[0][1])],
                            dst_v.at[pl.ds(0, STAGES[0][1])])
            pltpu.async_copy(h_ref.at[src_v.at[0]], b0, sem_g)
            pltpu.async_copy(h_ref.at[src_v.at[1]], b1, sem_g)
            pltpu.async_copy(h_ref.at[src_v.at[2]], b2, sem_g)

            zv = jnp.zeros((16,), jnp.float32)

            def zbody(i, carry):
                b3[i // 8, pl.ds((i % 8) * 16, 16)] = zv
                return carry

            lax.fori_loop(0, CHUNK * 8, zbody, 0)
            for r in range(Z_FULL):
                pltpu.sync_copy(b3, acc.at[pl.ds(row0 + r * ZROWS, ZROWS)])
            pltpu.sync_copy(b3.at[pl.ds(0, Z_REM)],
                            acc.at[pl.ds(row0 + Z_FULL * ZROWS, Z_REM)])

            @pl.when(s == 0)
            def _():
                pltpu.sync_copy(b3.at[pl.ds(0, ROW_TAIL)],
                                acc.at[pl.ds(N_SUB * ROWS_MAIN, ROW_TAIL)])

            plsc.subcore_barrier()
            def gather(j, b):
                pltpu.async_copy(h_ref.at[src_v.at[j]], bufs[b], sem_g)

            def gwait(j, b):
                pltpu.make_async_copy(
                    h_ref.at[src_v.at[j]], bufs[b], sem_g).wait()

            def scat(j, b):
                pltpu.async_copy(
                    bufs[b], acc.at[dst_v.at[j]], sem_s, add=True)

            def swait(j, b):
                pltpu.make_async_copy(
                    bufs[b], acc.at[dst_v.at[j]], sem_s).wait()

            def slot(j, b, n, traced):
                gwait(j, b)
                scat(j, b)
                swait(j - 1, (b - 1) % NBUF)
                if traced:
                    @pl.when(j + 3 < n)
                    def _():
                        gather(j + 3, (b + 3) % NBUF)
                elif j + 3 < n:
                    gather(j + 3, (b + 3) % NBUF)

            for stage_i, (chunk0, n) in enumerate(STAGES):
                if stage_i > 0:
                    pltpu.sync_copy(
                        ei_hbm.at[0, s, pl.ds(chunk0, n)],
                        src_v.at[pl.ds(0, n)])
                    pltpu.sync_copy(
                        ei_hbm.at[1, s, pl.ds(chunk0, n)],
                        dst_v.at[pl.ds(0, n)])
                    gather(0, 0)
                    gather(1, 1)
                    gather(2, 2)
                gwait(0, 0)
                scat(0, 0)
                gather(3, 3)

                n_fori = (n - 1) // NBUF

                def body(i, carry):
                    j0 = 1 + NBUF * i
                    for r in range(NBUF):
                        slot(j0 + r, (1 + r) % NBUF, n, traced=True)
                    return carry

                lax.fori_loop(0, n_fori, body, 0)
                for j in range(1 + NBUF * n_fori, n):
                    slot(j, j % NBUF, n, traced=False)
                swait(n - 1, (n - 1) % NBUF)

        @pl.when(c == 0)
        def _():
            run(h0_hbm)

        @pl.when(c == 1)
        def _():
            run(h1_hbm)

        plsc.subcore_barrier()

        def writeout(col0):
            pltpu.sync_copy(
                acc.at[pl.ds(row0, ROWS_MAIN)],
                out_hbm.at[pl.ds(row0, ROWS_MAIN), pl.ds(col0, HALF)])

            @pl.when(s == 0)
            def _():
                tail0 = N_SUB * ROWS_MAIN
                pltpu.sync_copy(
                    acc.at[pl.ds(tail0, ROW_TAIL)],
                    out_hbm.at[pl.ds(tail0, ROW_TAIL), pl.ds(col0, HALF)])

        @pl.when(c == 0)
        def _():
            writeout(0)

        @pl.when(c == 1)
        def _():
            writeout(HALF)

    return k(h0, h1, ei4)


def kernel(x, edge_index, W1, b1, W2, b2):
    ei4 = edge_index.astype(jnp.int32).reshape(2, N_SUB, N_CHUNKS, CHUNK)
    h0, h1 = _mlp(x, W1, b1.reshape(1, EMB), W2, b2.reshape(1, EMB))
    return _sc_message_pass(h0, h1, ei4)


# speedup vs baseline: 1.0384x; 1.0384x over previous
import functools

import jax
import jax.numpy as jnp
from jax import lax
from jax.experimental import pallas as pl
from jax.experimental.pallas import tpu as pltpu
from jax.experimental.pallas import tpu_sc as plsc

N_NODES = 10000
EMB = 256
HALF = 128
N_EDGES = 160000

N_SUB = 16
EDGES_PER_TILE = N_EDGES // N_SUB
CHUNK = 80
N_CHUNKS = EDGES_PER_TILE // CHUNK
STAGES = ((0, 32), (32, 32), (64, 32), (96, 29))
STAGE_MAX = 32
NBUF = 4
ROWS_MAIN = 624
ROW_TAIL = N_NODES - N_SUB * ROWS_MAIN
ZROWS = 80
Z_FULL = 7
Z_REM = ROWS_MAIN - Z_FULL * ZROWS

MLP_BLK = 5000


def _mlp_body(x_ref, w1_ref, b1_ref, w2_ref, b2_ref, h0_ref, h1_ref):
    x16 = x_ref[...].astype(jnp.bfloat16)
    h = jnp.dot(x16, w1_ref[...].astype(jnp.bfloat16),
                preferred_element_type=jnp.float32)
    h = jnp.maximum(h + b1_ref[...], 0.0).astype(jnp.bfloat16)
    h = jnp.dot(h, w2_ref[...].astype(jnp.bfloat16),
                preferred_element_type=jnp.float32)
    h = jnp.maximum(h + b2_ref[...], 0.0)
    h0_ref[...] = h[:, :HALF]
    h1_ref[...] = h[:, HALF:]


def _mlp(x, W1, b1, W2, b2):
    return pl.pallas_call(
        _mlp_body,
        grid=(N_NODES // MLP_BLK,),
        in_specs=[
            pl.BlockSpec((MLP_BLK, EMB), lambda i: (i, 0)),
            pl.BlockSpec((EMB, EMB), lambda i: (0, 0)),
            pl.BlockSpec((1, EMB), lambda i: (0, 0)),
            pl.BlockSpec((EMB, EMB), lambda i: (0, 0)),
            pl.BlockSpec((1, EMB), lambda i: (0, 0)),
        ],
        out_specs=[
            pl.BlockSpec((MLP_BLK, HALF), lambda i: (i, 0)),
            pl.BlockSpec((MLP_BLK, HALF), lambda i: (i, 0)),
        ],
        out_shape=[
            jax.ShapeDtypeStruct((N_NODES, HALF), jnp.float32),
            jax.ShapeDtypeStruct((N_NODES, HALF), jnp.float32),
        ],
    )(x, W1, b1, W2, b2)


def _sc_message_pass(h0, h1, ei4):
    mesh = plsc.VectorSubcoreMesh(core_axis_name="c", subcore_axis_name="s")

    @functools.partial(
        pl.kernel,
        mesh=mesh,
        out_type=jax.ShapeDtypeStruct((N_NODES, EMB), jnp.float32),
        scratch_types=[
            pltpu.VMEM((2, STAGE_MAX, CHUNK), jnp.int32),
            pltpu.VMEM((CHUNK, HALF), jnp.float32),
            pltpu.VMEM((CHUNK, HALF), jnp.float32),
            pltpu.VMEM((CHUNK, HALF), jnp.float32),
            pltpu.VMEM((CHUNK, HALF), jnp.float32),
            pltpu.VMEM_SHARED((N_NODES, HALF), jnp.float32),
            pltpu.SemaphoreType.DMA,
            pltpu.SemaphoreType.DMA,
        ],
    )
    def k(h0_hbm, h1_hbm, ei_hbm, out_hbm, idx_v, b0, b1, b2, b3,
          acc, sem_g, sem_s):
        c = lax.axis_index("c")
        s = lax.axis_index("s")
        row0 = s * ROWS_MAIN
        bufs = (b0, b1, b2, b3)

        def run(h_ref):
            pltpu.sync_copy(ei_hbm.at[:, s, pl.ds(0, STAGES[0][1])],
                            idx_v.at[:, pl.ds(0, STAGES[0][1])])
            pltpu.async_copy(h_ref.at[idx_v.at[0, 0]], b0, sem_g)
            pltpu.async_copy(h_ref.at[idx_v.at[0, 1]], b1, sem_g)
            pltpu.async_copy(h_ref.at[idx_v.at[0, 2]], b2, sem_g)

            zv = jnp.zeros((16,), jnp.float32)

            def zbody(i, carry):
                b3[i // 8, pl.ds((i % 8) * 16, 16)] = zv
                return carry

            lax.fori_loop(0, CHUNK * 8, zbody, 0)
            for r in range(Z_FULL):
                pltpu.sync_copy(b3, acc.at[pl.ds(row0 + r * ZROWS, ZROWS)])
            pltpu.sync_copy(b3.at[pl.ds(0, Z_REM)],
                            acc.at[pl.ds(row0 + Z_FULL * ZROWS, Z_REM)])

            @pl.when(s == 0)
            def _():
                pltpu.sync_copy(b3.at[pl.ds(0, ROW_TAIL)],
                                acc.at[pl.ds(N_SUB * ROWS_MAIN, ROW_TAIL)])

            plsc.subcore_barrier()
            def gather(j, b):
                pltpu.async_copy(h_ref.at[idx_v.at[0, j]], bufs[b], sem_g)

            def gwait(j, b):
                pltpu.make_async_copy(
                    h_ref.at[idx_v.at[0, j]], bufs[b], sem_g).wait()

            def scat(j, b):
                pltpu.async_copy(
                    bufs[b], acc.at[idx_v.at[1, j]], sem_s, add=True)

            def swait(j, b):
                pltpu.make_async_copy(
                    bufs[b], acc.at[idx_v.at[1, j]], sem_s).wait()

            def slot(j, b, n, traced):
                gwait(j, b)
                scat(j, b)
                swait(j - 1, (b - 1) % NBUF)
                if traced:
                    @pl.when(j + 3 < n)
                    def _():
                        gather(j + 3, (b + 3) % NBUF)
                elif j + 3 < n:
                    gather(j + 3, (b + 3) % NBUF)

            for stage_i, (chunk0, n) in enumerate(STAGES):
                if stage_i > 0:
                    pltpu.sync_copy(
                        ei_hbm.at[:, s, pl.ds(chunk0, n)],
                        idx_v.at[:, pl.ds(0, n)])
                    gather(0, 0)
                    gather(1, 1)
                    gather(2, 2)
                gwait(0, 0)
                scat(0, 0)
                gather(3, 3)

                n_fori = (n - 1) // NBUF

                def body(i, carry):
                    j0 = 1 + NBUF * i
                    for r in range(NBUF):
                        slot(j0 + r, (1 + r) % NBUF, n, traced=True)
                    return carry

                lax.fori_loop(0, n_fori, body, 0)
                for j in range(1 + NBUF * n_fori, n):
                    slot(j, j % NBUF, n, traced=False)
                swait(n - 1, (n - 1) % NBUF)

        @pl.when(c == 0)
        def _():
            run(h0_hbm)

        @pl.when(c == 1)
        def _():
            run(h1_hbm)

        plsc.subcore_barrier()

        def writeout(col0):
            pltpu.sync_copy(
                acc.at[pl.ds(row0, ROWS_MAIN)],
                out_hbm.at[pl.ds(row0, ROWS_MAIN), pl.ds(col0, HALF)])

            @pl.when(s == 0)
            def _():
                tail0 = N_SUB * ROWS_MAIN
                pltpu.sync_copy(
                    acc.at[pl.ds(tail0, ROW_TAIL)],
                    out_hbm.at[pl.ds(tail0, ROW_TAIL), pl.ds(col0, HALF)])

        @pl.when(c == 0)
        def _():
            writeout(0)

        @pl.when(c == 1)
        def _():
            writeout(HALF)

    return k(h0, h1, ei4)


def kernel(x, edge_index, W1, b1, W2, b2):
    ei4 = edge_index.astype(jnp.int32).reshape(2, N_SUB, N_CHUNKS, CHUNK)
    h0, h1 = _mlp(x, W1, b1.reshape(1, EMB), W2, b2.reshape(1, EMB))
    return _sc_message_pass(h0, h1, ei4)
